# baseline (speedup 1.0000x reference)
import functools

import jax
import jax.numpy as jnp
from jax import lax
from jax.experimental import pallas as pl
from jax.experimental.pallas import tpu as pltpu
from jax.experimental.pallas import tpu_sc as plsc

_NUM_CORES = 2
_NUM_SUBCORES = 16
_NUM_WORKERS = _NUM_CORES * _NUM_SUBCORES

_GROUP = 4

_ROW = 128


def _gather_kernel(batch, length, emb, table_hbm, idx_hbm, out_hbm,
                   idx_v, buf0, buf1, sem0, sem1):
    rows_w = batch // _NUM_WORKERS
    n_groups = rows_w // _GROUP
    wid = lax.axis_index("s") * _NUM_CORES + lax.axis_index("c")
    row_base = wid * rows_w

    pltpu.sync_copy(idx_hbm.at[pl.ds(row_base, rows_w)], idx_v)

    def fire(buf, sem, g):
        for j in range(_GROUP):
            pltpu.async_copy(
                table_hbm.at[idx_v.at[g * _GROUP + j]],
                buf.at[j],
                sem,
            )

    def drain(buf, sem):
        for j in range(_GROUP):
            pltpu.make_async_copy(
                table_hbm.at[idx_v.at[j]], buf.at[j], sem).wait()

    def writeback(buf, g):
        pltpu.sync_copy(
            buf.at[:, :, pl.ds(0, emb)],
            out_hbm.at[pl.ds(row_base + g * _GROUP, _GROUP)])

    fire(buf0, sem0, 0)
    fire(buf1, sem1, 1)

    def body(t, carry):
        g0 = 2 * t
        g1 = g0 + 1

        drain(buf0, sem0)
        writeback(buf0, g0)

        @pl.when(g0 + 2 < n_groups)
        def _():
            fire(buf0, sem0, g0 + 2)

        drain(buf1, sem1)
        writeback(buf1, g1)

        @pl.when(g1 + 2 < n_groups)
        def _():
            fire(buf1, sem1, g1 + 2)

        return carry

    lax.fori_loop(0, n_groups // 2, body, 0)


def kernel(indices, table):
    batch, length = indices.shape
    emb = table.shape[1]
    rows_w = batch // _NUM_WORKERS

    idx = indices.astype(jnp.int32)
    table_pad = jnp.pad(table, ((0, 0), (0, _ROW - emb)))

    mesh = plsc.VectorSubcoreMesh(core_axis_name="c", subcore_axis_name="s")
    out = pl.kernel(
        functools.partial(_gather_kernel, batch, length, emb),
        mesh=mesh,
        out_type=jax.ShapeDtypeStruct((batch, length, emb), jnp.float32),
        scratch_types=[
            pltpu.VMEM((rows_w, length), jnp.int32),
            pltpu.VMEM((_GROUP, length, _ROW), jnp.float32),
            pltpu.VMEM((_GROUP, length, _ROW), jnp.float32),
            pltpu.SemaphoreType.DMA,
            pltpu.SemaphoreType.DMA,
        ],
        compiler_params=pltpu.CompilerParams(use_tc_tiling_on_sc=False),
    )(table_pad, idx)
    return out


# speedup vs baseline: 1.0630x; 1.0630x over previous
import functools

import jax
import jax.numpy as jnp
from jax import lax
from jax.experimental import pallas as pl
from jax.experimental.pallas import tpu as pltpu
from jax.experimental.pallas import tpu_sc as plsc

_NUM_CORES = 2
_NUM_SUBCORES = 16
_NUM_WORKERS = _NUM_CORES * _NUM_SUBCORES

_GROUP = 2

_ROW = 128
_VREG = 16


def _gather_kernel(batch, length, emb, table_hbm, idx_hbm, out_hbm,
                   idx_v, buf0, buf1, cbuf, sem0, sem1):
    rows_w = batch // _NUM_WORKERS
    n_groups = rows_w // _GROUP
    wid = lax.axis_index("s") * _NUM_CORES + lax.axis_index("c")
    row_base = wid * rows_w

    pltpu.sync_copy(idx_hbm.at[pl.ds(row_base, rows_w)], idx_v)

    def fire(buf, sem, g):
        for j in range(_GROUP):
            pltpu.async_copy(
                table_hbm.at[idx_v.at[g * _GROUP + j]],
                buf.at[j],
                sem,
            )

    def drain(buf, sem):
        for j in range(_GROUP):
            pltpu.make_async_copy(
                table_hbm.at[idx_v.at[j]], buf.at[j], sem).wait()

    def compact(buf):
        for j in range(_GROUP):
            for r in range(length):
                for c in range(emb // _VREG):
                    cbuf[j, r, pl.ds(c * _VREG, _VREG)] = (
                        buf[j, r, pl.ds(c * _VREG, _VREG)])

    def writeback(g):
        pltpu.sync_copy(
            cbuf, out_hbm.at[pl.ds(row_base + g * _GROUP, _GROUP)])

    fire(buf0, sem0, 0)
    fire(buf1, sem1, 1)

    def body(t, carry):
        g0 = 2 * t
        g1 = g0 + 1

        drain(buf0, sem0)
        compact(buf0)

        @pl.when(g0 + 2 < n_groups)
        def _():
            fire(buf0, sem0, g0 + 2)

        writeback(g0)

        drain(buf1, sem1)
        compact(buf1)

        @pl.when(g1 + 2 < n_groups)
        def _():
            fire(buf1, sem1, g1 + 2)

        writeback(g1)

        return carry

    lax.fori_loop(0, n_groups // 2, body, 0)


def kernel(indices, table):
    batch, length = indices.shape
    emb = table.shape[1]
    rows_w = batch // _NUM_WORKERS

    idx = indices.astype(jnp.int32)
    table_pad = jnp.pad(table, ((0, 0), (0, _ROW - emb)))

    mesh = plsc.VectorSubcoreMesh(core_axis_name="c", subcore_axis_name="s")
    out = pl.kernel(
        functools.partial(_gather_kernel, batch, length, emb),
        mesh=mesh,
        out_type=jax.ShapeDtypeStruct((batch, length, emb), jnp.float32),
        scratch_types=[
            pltpu.VMEM((rows_w, length), jnp.int32),
            pltpu.VMEM((_GROUP, length, _ROW), jnp.float32),
            pltpu.VMEM((_GROUP, length, _ROW), jnp.float32),
            pltpu.VMEM((_GROUP, length, emb), jnp.float32),
            pltpu.SemaphoreType.DMA,
            pltpu.SemaphoreType.DMA,
        ],
        compiler_params=pltpu.CompilerParams(use_tc_tiling_on_sc=True),
    )(table_pad, idx)
    return out
